# baseline (speedup 1.0000x reference)
import jax
import jax.numpy as jnp
import numpy as np
from jax.experimental import pallas as pl
from jax.experimental.pallas import tpu as pltpu
from jax.experimental.pallas import tpu_sc as plsc
from jax.experimental.shard_map import shard_map
from jax.sharding import Mesh, NamedSharding, PartitionSpec as P

B, S, D, N = 8, 50, 128, 100000
GW = 128

_DEVS = jax.devices()
_NDEV = 2 if len(_DEVS) >= 2 and N % 2 == 0 else 1
_MESH = Mesh(np.array(_DEVS[:_NDEV]), ("x",))
NSH = N // _NDEV
BNS = ((NSH + 127) // 128) * 128

_INTERPRET = False


def _sc_gather(table, flat_idx):
    n_idx = flat_idx.shape[0]
    idx2 = flat_idx.reshape(1, n_idx)
    mesh = plsc.VectorSubcoreMesh(core_axis_name="core",
                                  subcore_axis_name="subcore")

    @pl.kernel(out_type=jax.ShapeDtypeStruct((n_idx, D), table.dtype),
               mesh=mesh)
    def gather_kernel(x_hbm, i_hbm, o_hbm):
        def body(i_vmem, o_vmem):
            pltpu.sync_copy(x_hbm.at[i_vmem.at[0]], o_vmem)

        pltpu.emit_pipeline(
            body,
            grid=(n_idx // GW,),
            in_specs=[pl.BlockSpec((1, GW), index_map=lambda i: (0, i))],
            out_specs=[pl.BlockSpec((GW, D), index_map=lambda i: (i, 0))],
            core_axis_name=("core", "subcore"),
            dimension_semantics=(pltpu.PARALLEL,),
        )(i_hbm, o_hbm)

    return gather_kernel(table, idx2)


def _val2_kernel(seq_row_ref, seq_col_ref, hidden_ref, sel_ref, wt_ref,
                 bias_ref, val2_ref):
    seq_r = seq_row_ref[0]
    seq_c = seq_col_ref[0]
    eq = (seq_c == seq_r).astype(jnp.float32)
    ii = jax.lax.broadcasted_iota(jnp.int32, (S, S), 0)
    jj = jax.lax.broadcasted_iota(jnp.int32, (S, S), 1)
    tril = (ii >= jj).astype(jnp.float32)
    counts = jnp.dot(tril, eq, preferred_element_type=jnp.float32)
    tcf = jnp.log2(counts + 1.0)
    tot = jnp.sum(eq, axis=0, keepdims=True)
    emb = jnp.dot(hidden_ref[0], wt_ref[...],
                  preferred_element_type=jnp.float32) + bias_ref[...]
    logits = jax.lax.dot_general(emb, sel_ref[0], (((1,), (1,)), ((), ())),
                                 preferred_element_type=jnp.float32)
    lt = logits * tcf
    val2 = jnp.dot(lt, eq, preferred_element_type=jnp.float32)
    val2_ref[0] = val2 / tot


def _scatter_kernel(seq_ref, val2_ref, out_ref):
    b = pl.program_id(0)
    out_ref[...] = jnp.zeros_like(out_ref)
    lane_iota = jax.lax.broadcasted_iota(jnp.int32, (S, 128), 1)
    for j in range(S):
        c = seq_ref[b, j]
        valid = jnp.logical_and(c >= 0, c < NSH)
        cl = jnp.clip(c, 0, NSH - 1)
        cw = (cl // 128) * 128
        lane = cl - cw
        window = out_ref[0, :, pl.ds(cw, 128)]
        patch = jnp.where(jnp.logical_and(lane_iota == lane, valid),
                          val2_ref[0, :, j:j + 1], window)
        out_ref[0, :, pl.ds(cw, 128)] = patch


def _per_device(seq, hidden_states, item_embeddings, wt, brow):
    n_pad = ((B * S + GW - 1) // GW) * GW
    flat_idx = jnp.concatenate(
        [seq.reshape(-1),
         jnp.zeros((n_pad - B * S,), jnp.int32)])
    sel = _sc_gather(item_embeddings, flat_idx)[:B * S].reshape(B, S, D)

    val2 = pl.pallas_call(
        _val2_kernel,
        grid=(B,),
        in_specs=[
            pl.BlockSpec((1, 1, S), lambda b: (b, 0, 0)),
            pl.BlockSpec((1, S, 1), lambda b: (b, 0, 0)),
            pl.BlockSpec((1, S, D), lambda b: (b, 0, 0)),
            pl.BlockSpec((1, S, D), lambda b: (b, 0, 0)),
            pl.BlockSpec((D, D), lambda b: (0, 0)),
            pl.BlockSpec((1, D), lambda b: (0, 0)),
        ],
        out_specs=pl.BlockSpec((1, S, S), lambda b: (b, 0, 0)),
        out_shape=jax.ShapeDtypeStruct((B, S, S), jnp.float32),
        interpret=_INTERPRET,
    )(
        seq.reshape(B, 1, S),
        seq.reshape(B, S, 1),
        hidden_states,
        sel,
        wt,
        brow,
    )

    kidx = jax.lax.axis_index("x").astype(jnp.int32)
    seq_local = seq - kidx * NSH

    out = pl.pallas_call(
        _scatter_kernel,
        grid=(B,),
        in_specs=[
            pl.BlockSpec(memory_space=pltpu.SMEM),
            pl.BlockSpec((1, S, S), lambda b: (b, 0, 0)),
        ],
        out_specs=pl.BlockSpec((1, S, BNS), lambda b: (b, 0, 0)),
        out_shape=jax.ShapeDtypeStruct((B, S, NSH), jnp.float32),
        interpret=_INTERPRET,
    )(seq_local, val2)
    return out


_sharded = shard_map(
    _per_device,
    mesh=_MESH,
    in_specs=(P(), P(), P(), P(), P()),
    out_specs=P(None, None, "x"),
    check_rep=False,
)


@jax.jit
def kernel(input_seq, hidden_states, item_embeddings, W_emb, b_emb):
    seq = input_seq.astype(jnp.int32)
    return _sharded(seq, hidden_states, item_embeddings, W_emb.T,
                    b_emb.reshape(1, D))


# speedup vs baseline: 1.2007x; 1.2007x over previous
import jax
import jax.numpy as jnp
from jax.experimental import pallas as pl
from jax.experimental.pallas import tpu as pltpu
from jax.experimental.pallas import tpu_sc as plsc

B, S, D, N = 8, 50, 128, 100000
BN = 100096
GW = 128

_INTERPRET = False


def _sc_gather(table, flat_idx):
    n_idx = flat_idx.shape[0]
    idx2 = flat_idx.reshape(1, n_idx)
    mesh = plsc.VectorSubcoreMesh(core_axis_name="core",
                                  subcore_axis_name="subcore")

    @pl.kernel(out_type=jax.ShapeDtypeStruct((n_idx, D), table.dtype),
               mesh=mesh)
    def gather_kernel(x_hbm, i_hbm, o_hbm):
        def body(i_vmem, o_vmem):
            pltpu.sync_copy(x_hbm.at[i_vmem.at[0]], o_vmem)

        pltpu.emit_pipeline(
            body,
            grid=(n_idx // GW,),
            in_specs=[pl.BlockSpec((1, GW), index_map=lambda i: (0, i))],
            out_specs=[pl.BlockSpec((GW, D), index_map=lambda i: (i, 0))],
            core_axis_name=("core", "subcore"),
            dimension_semantics=(pltpu.PARALLEL,),
        )(i_hbm, o_hbm)

    return gather_kernel(table, idx2)


def _fused_kernel(seq_smem_ref, seq_row_ref, seq_col_ref, hidden_ref,
                  sel_ref, wt_ref, bias_ref, out_ref):
    b = pl.program_id(0)
    seq_r = seq_row_ref[0]
    seq_c = seq_col_ref[0]
    eq = (seq_c == seq_r).astype(jnp.float32)
    ii = jax.lax.broadcasted_iota(jnp.int32, (S, S), 0)
    jj = jax.lax.broadcasted_iota(jnp.int32, (S, S), 1)
    tril = (ii >= jj).astype(jnp.float32)
    counts = jnp.dot(tril, eq, preferred_element_type=jnp.float32)
    tcf = jnp.log2(counts + 1.0)
    tot = jnp.sum(eq, axis=0, keepdims=True)
    emb = jnp.dot(hidden_ref[0], wt_ref[...],
                  preferred_element_type=jnp.float32) + bias_ref[...]
    logits = jax.lax.dot_general(emb, sel_ref[0], (((1,), (1,)), ((), ())),
                                 preferred_element_type=jnp.float32)
    lt = logits * tcf
    val2 = jnp.dot(lt, eq, preferred_element_type=jnp.float32) / tot

    out_ref[...] = jnp.zeros_like(out_ref)
    lane_iota = jax.lax.broadcasted_iota(jnp.int32, (S, 128), 1)
    for j in range(S):
        c = seq_smem_ref[b, j]
        cw = (c // 128) * 128
        lane = c - cw
        window = out_ref[0, :, pl.ds(cw, 128)]
        patch = jnp.where(lane_iota == lane, val2[:, j:j + 1], window)
        out_ref[0, :, pl.ds(cw, 128)] = patch


@jax.jit
def kernel(input_seq, hidden_states, item_embeddings, W_emb, b_emb):
    seq = input_seq.astype(jnp.int32)
    n_pad = ((B * S + GW - 1) // GW) * GW
    flat_idx = jnp.concatenate(
        [seq.reshape(-1),
         jnp.zeros((n_pad - B * S,), jnp.int32)])
    sel = _sc_gather(item_embeddings, flat_idx)[:B * S].reshape(B, S, D)

    out = pl.pallas_call(
        _fused_kernel,
        grid=(B,),
        in_specs=[
            pl.BlockSpec(memory_space=pltpu.SMEM),
            pl.BlockSpec((1, 1, S), lambda b: (b, 0, 0)),
            pl.BlockSpec((1, S, 1), lambda b: (b, 0, 0)),
            pl.BlockSpec((1, S, D), lambda b: (b, 0, 0)),
            pl.BlockSpec((1, S, D), lambda b: (b, 0, 0)),
            pl.BlockSpec((D, D), lambda b: (0, 0)),
            pl.BlockSpec((1, D), lambda b: (0, 0)),
        ],
        out_specs=pl.BlockSpec((1, S, BN), lambda b: (b, 0, 0)),
        out_shape=jax.ShapeDtypeStruct((B, S, N), jnp.float32),
        interpret=_INTERPRET,
    )(
        seq,
        seq.reshape(B, 1, S),
        seq.reshape(B, S, 1),
        hidden_states,
        sel,
        W_emb.T,
        b_emb.reshape(1, D),
    )
    return out


# speedup vs baseline: 1.2147x; 1.0116x over previous
import jax
import jax.numpy as jnp
from jax.experimental import pallas as pl
from jax.experimental.pallas import tpu as pltpu
from jax.experimental.pallas import tpu_sc as plsc

B, S, D, N = 8, 50, 128, 100000
BN = 100096
GW = 128

_INTERPRET = False


def _sc_gather(table, flat_idx):
    n_idx = flat_idx.shape[0]
    idx2 = flat_idx.reshape(1, n_idx)
    mesh = plsc.VectorSubcoreMesh(core_axis_name="core",
                                  subcore_axis_name="subcore")

    @pl.kernel(out_type=jax.ShapeDtypeStruct((n_idx, D), table.dtype),
               mesh=mesh)
    def gather_kernel(x_hbm, i_hbm, o_hbm):
        def body(i_vmem, o_vmem):
            pltpu.sync_copy(x_hbm.at[i_vmem.at[0]], o_vmem)

        pltpu.emit_pipeline(
            body,
            grid=(n_idx // GW,),
            in_specs=[pl.BlockSpec((1, GW), index_map=lambda i: (0, i))],
            out_specs=[pl.BlockSpec((GW, D), index_map=lambda i: (i, 0))],
            core_axis_name=("core", "subcore"),
            dimension_semantics=(pltpu.PARALLEL,),
        )(i_hbm, o_hbm)

    return gather_kernel(table, idx2)


def _fused_kernel(seq_smem_ref, seq_row_ref, seq_col_ref, hidden_ref,
                  sel_ref, wt_ref, bias_ref, out_ref):
    b = pl.program_id(0)
    seq_r = seq_row_ref[0]
    seq_c = seq_col_ref[0]
    eq = (seq_c == seq_r).astype(jnp.float32)
    ii = jax.lax.broadcasted_iota(jnp.int32, (S, S), 0)
    jj = jax.lax.broadcasted_iota(jnp.int32, (S, S), 1)
    tril = (ii >= jj).astype(jnp.float32)
    counts = jnp.dot(tril, eq, preferred_element_type=jnp.float32)
    tcf = jnp.log2(counts + 1.0)
    tot = jnp.sum(eq, axis=0, keepdims=True)
    emb = jax.lax.dot_general(hidden_ref[0], wt_ref[...],
                              (((1,), (1,)), ((), ())),
                              preferred_element_type=jnp.float32)
    emb = emb + bias_ref[...]
    sel = sel_ref[0][:S]
    logits = jax.lax.dot_general(emb, sel, (((1,), (1,)), ((), ())),
                                 preferred_element_type=jnp.float32)
    lt = logits * tcf
    val2 = jnp.dot(lt, eq, preferred_element_type=jnp.float32) / tot

    out_ref[...] = jnp.zeros_like(out_ref)
    lane_iota = jax.lax.broadcasted_iota(jnp.int32, (S, 128), 1)
    for j in range(S):
        c = seq_smem_ref[b, j]
        cw = (c // 128) * 128
        lane = c - cw
        window = out_ref[0, :, pl.ds(cw, 128)]
        patch = jnp.where(lane_iota == lane, val2[:, j:j + 1], window)
        out_ref[0, :, pl.ds(cw, 128)] = patch


@jax.jit
def kernel(input_seq, hidden_states, item_embeddings, W_emb, b_emb):
    seq = input_seq.astype(jnp.int32)
    SP = 64
    idx_pad = jnp.concatenate(
        [seq, jnp.zeros((B, SP - S), jnp.int32)], axis=1)
    sel = _sc_gather(item_embeddings, idx_pad.reshape(-1)).reshape(B, SP, D)

    out = pl.pallas_call(
        _fused_kernel,
        grid=(B,),
        in_specs=[
            pl.BlockSpec(memory_space=pltpu.SMEM),
            pl.BlockSpec((1, 1, S), lambda b: (b, 0, 0)),
            pl.BlockSpec((1, S, 1), lambda b: (b, 0, 0)),
            pl.BlockSpec((1, S, D), lambda b: (b, 0, 0)),
            pl.BlockSpec((1, 64, D), lambda b: (b, 0, 0)),
            pl.BlockSpec((D, D), lambda b: (0, 0)),
            pl.BlockSpec((1, D), lambda b: (0, 0)),
        ],
        out_specs=pl.BlockSpec((1, S, BN), lambda b: (b, 0, 0)),
        out_shape=jax.ShapeDtypeStruct((B, S, N), jnp.float32),
        interpret=_INTERPRET,
    )(
        seq,
        seq.reshape(B, 1, S),
        seq.reshape(B, S, 1),
        hidden_states,
        sel,
        W_emb,
        b_emb.reshape(1, D),
    )
    return out
